# baseline (speedup 1.0000x reference)
import functools

import jax
import jax.numpy as jnp
from jax import lax
from jax.experimental import pallas as pl
from jax.experimental.pallas import tpu as pltpu
from jax.experimental.pallas import tpu_sc as plsc

_NUM_CORES = 2
_NUM_SUBCORES = 16
_NUM_WORKERS = _NUM_CORES * _NUM_SUBCORES


def _gather_body(rows_per_worker,
                 idx0_hbm, idx1_hbm, table_hbm,
                 out0_hbm, out1_hbm,
                 idx_v, vals_v, sem):
    cid = lax.axis_index("c")
    sid = lax.axis_index("s")
    base = sid * rows_per_worker
    sl = pl.ds(base, rows_per_worker)

    @pl.when(cid == 0)
    def _():
        pltpu.sync_copy(idx0_hbm.at[sl], idx_v)
        pltpu.async_copy(table_hbm.at[idx_v], vals_v, sem).wait()
        pltpu.sync_copy(vals_v, out0_hbm.at[sl])

    @pl.when(cid == 1)
    def _():
        pltpu.sync_copy(idx1_hbm.at[sl], idx_v)
        pltpu.async_copy(table_hbm.at[idx_v], vals_v, sem).wait()
        pltpu.sync_copy(vals_v, out1_hbm.at[sl])


def kernel(item_id, batch_size, item_bias):
    b, n = item_id.shape
    rows_per_worker = b // _NUM_SUBCORES
    table = item_bias[:, 0]
    idx0 = item_id[:, 0]
    idx1 = item_id[:, 1]

    mesh = plsc.VectorSubcoreMesh(core_axis_name="c", subcore_axis_name="s")
    out0, out1 = pl.kernel(
        functools.partial(_gather_body, rows_per_worker),
        out_type=(
            jax.ShapeDtypeStruct((b,), jnp.float32),
            jax.ShapeDtypeStruct((b,), jnp.float32),
        ),
        mesh=mesh,
        scratch_types=[
            pltpu.VMEM((rows_per_worker,), jnp.int32),
            pltpu.VMEM((rows_per_worker,), jnp.float32),
            pltpu.SemaphoreType.DMA,
        ],
    )(idx0, idx1, table)
    return jnp.stack([out0, out1], axis=-1)
